# baseline (speedup 1.0000x reference)
import functools

import jax
import jax.numpy as jnp
from jax import lax
from jax.experimental import pallas as pl
from jax.experimental.pallas import tpu as pltpu
from jax.experimental.pallas import tpu_sc as plsc

B = 16384
NF = 26
NCONT = 13
D = 32
FT = NF + NCONT
V = 2600012
NW = 32
RW = B // NW
CH = 128
NCH = RW // CH
NBLK = B // CH


def _body(xc_hbm, xnum_hbm, tf_hbm, w_hbm, b_hbm, out_hbm,
          xblk_v, idxt_v, fidx_v, dtile_v, xnum_v, w_v, b_v, ntile_v,
          gsem0, gsem1, gsem2, gsem3, wsem0, wsem1, wsem2, wsem3,
          nsem0, nsem1):
    gsem = (gsem0, gsem1, gsem2, gsem3)
    wsem = (wsem0, wsem1, wsem2, wsem3)
    nsem = (nsem0, nsem1)

    cid = lax.axis_index("c")
    sid = lax.axis_index("s")
    wid = sid * 2 + cid
    b0 = wid * RW
    blk0 = wid * NCH

    pltpu.sync_copy(w_hbm, w_v)
    pltpu.sync_copy(b_hbm, b_v)
    pltpu.sync_copy(xnum_hbm.at[pl.ds(b0, RW)], xnum_v)

    lanes = lax.iota(jnp.int32, 16)

    def stage(c, _):
        pltpu.sync_copy(xc_hbm.at[pl.ds(b0 + c * CH, CH)], xblk_v)
        for f in range(NF):
            fcol = jnp.full((16,), f, jnp.int32)
            for g in range(CH // 16):
                v = plsc.load_gather(xblk_v, [g * 16 + lanes, fcol])
                idxt_v[f * NCH + c, pl.ds(g * 16, 16)] = v
        return 0

    lax.fori_loop(0, NCH, stage, 0)

    def field(k, _):
        for j in range(NCH):
            for g in range(CH // 16):
                tok = idxt_v[k * NCH + j, pl.ds(g * 16, 16)]
                for d in range(D):
                    fidx_v[j, pl.ds(d * CH + g * 16, 16)] = tok + d * V
        gathers = [
            pltpu.async_copy(
                tf_hbm.at[fidx_v.at[j]], dtile_v.at[j], gsem[j])
            for j in range(NCH)
        ]
        writes = []
        for j in range(NCH):
            gathers[j].wait()
            for db in range(4):
                writes.append(pltpu.async_copy(
                    dtile_v.at[j, pl.ds(db * 8 * CH, 8 * CH)],
                    out_hbm.at[k, db, blk0 + j],
                    wsem[j]))
        for wcp in writes:
            wcp.wait()
        return 0

    lax.fori_loop(0, NF, field, 0)

    def numeric(m, _):
        nwr = []
        for su in range(2):
            u = m * 2 + su
            c = u // NCONT
            n = u % NCONT
            ncol = jnp.full((16,), 1, jnp.int32) * n
            xv = [
                plsc.load_gather(
                    xnum_v, [c * CH + g * 16 + lanes, ncol])
                for g in range(CH // 16)
            ]
            for d in range(D):
                wsp = plsc.load_gather(
                    w_v, [ncol, jnp.full((16,), d, jnp.int32)])
                bsp = plsc.load_gather(
                    b_v, [ncol, jnp.full((16,), d, jnp.int32)])
                for g in range(CH // 16):
                    ntile_v[su, pl.ds(d * CH + g * 16, 16)] = \
                        xv[g] * wsp + bsp
            for db in range(4):
                nwr.append(pltpu.async_copy(
                    ntile_v.at[su, pl.ds(db * 8 * CH, 8 * CH)],
                    out_hbm.at[NF + n, db, blk0 + c],
                    nsem[su]))
        for wcp in nwr:
            wcp.wait()
        return 0

    lax.fori_loop(0, NCH * NCONT // 2, numeric, 0)


_embed = functools.partial(
    pl.kernel,
    out_type=jax.ShapeDtypeStruct((FT, 4, NBLK, 8 * CH), jnp.float32),
    mesh=plsc.VectorSubcoreMesh(core_axis_name="c", subcore_axis_name="s"),
    compiler_params=pltpu.CompilerParams(
        use_tc_tiling_on_sc=False, needs_layout_passes=False
    ),
    scratch_types=[
        pltpu.VMEM((CH, NF), jnp.int32),
        pltpu.VMEM((NF * NCH, CH), jnp.int32),
        pltpu.VMEM((NCH, D * CH), jnp.int32),
        pltpu.VMEM((NCH, D * CH), jnp.float32),
        pltpu.VMEM((RW, NCONT), jnp.float32),
        pltpu.VMEM((NCONT, D), jnp.float32),
        pltpu.VMEM((NCONT, D), jnp.float32),
        pltpu.VMEM((2, D * CH), jnp.float32),
    ] + [pltpu.SemaphoreType.DMA] * 10,
)(_body)


def kernel(x_categ, x_numer, embed_table, num_weights, num_biases):
    tflat = embed_table.T.reshape(-1)
    out5 = _embed(x_categ.astype(jnp.int32), x_numer, tflat,
                  num_weights, num_biases)
    out5 = out5.reshape(FT, 4, NBLK, 8, CH)
    return jnp.transpose(out5, (2, 4, 0, 1, 3)).reshape(B, FT, D)


# speedup vs baseline: 2.0154x; 2.0154x over previous
import functools

import jax
import jax.numpy as jnp
from jax import lax
from jax.experimental import pallas as pl
from jax.experimental.pallas import tpu as pltpu
from jax.experimental.pallas import tpu_sc as plsc

B = 16384
NF = 26
NCONT = 13
D = 32
FT = NF + NCONT
V = 2600012
VL = 650112
NW = 32
RW = B // NW
CH = 128
NCH = RW // CH
NBLK = B // CH
TROWS = 128
TGRID = VL // TROWS
TTOK = TROWS * 4


def _tc_body(src_ref, dst_ref):
    x = src_ref[...]
    for q in range(4):
        xq = lax.slice(x, (0, q * TROWS), (D, (q + 1) * TROWS))
        dst_ref[:, pl.ds(q * D, D)] = jnp.swapaxes(xq, 0, 1)


_transpose = pl.pallas_call(
    _tc_body,
    grid=(TGRID,),
    in_specs=[pl.BlockSpec((D, TTOK), lambda i: (0, i))],
    out_specs=pl.BlockSpec((TROWS, 128), lambda i: (i, 0)),
    out_shape=jax.ShapeDtypeStruct((VL, 128), jnp.float32),
    compiler_params=pltpu.CompilerParams(
        dimension_semantics=("arbitrary",)),
)


def _body(xc_hbm, xnum_hbm, t4_hbm, w_hbm, b_hbm, out_hbm,
          xblk_v, idxt_v, fidx_v, grows_v, dtile_v, xnum_v, w_v, b_v,
          ntile_v, gsem0, gsem1, gsem2, gsem3,
          wsem0, wsem1, wsem2, wsem3, nsem0, nsem1):
    gsem = (gsem0, gsem1, gsem2, gsem3)
    wsem = (wsem0, wsem1, wsem2, wsem3)
    nsem = (nsem0, nsem1)

    cid = lax.axis_index("c")
    sid = lax.axis_index("s")
    wid = sid * 2 + cid
    b0 = wid * RW
    blk0 = wid * NCH

    pltpu.sync_copy(w_hbm, w_v)
    pltpu.sync_copy(b_hbm, b_v)
    pltpu.sync_copy(xnum_hbm.at[pl.ds(b0, RW)], xnum_v)

    lanes = lax.iota(jnp.int32, 16)

    def stage(c, _):
        pltpu.sync_copy(xc_hbm.at[pl.ds(b0 + c * CH, CH)], xblk_v)
        for f in range(NF):
            fcol = jnp.full((16,), f, jnp.int32)
            for g in range(CH // 16):
                v = plsc.load_gather(xblk_v, [g * 16 + lanes, fcol])
                idxt_v[f * NCH + c, pl.ds(g * 16, 16)] = v
        return 0

    lax.fori_loop(0, NCH, stage, 0)

    def quad(m, _):
        steps = [m * 4 + su for su in range(4)]
        for su in range(4):
            s = steps[su]
            for g in range(CH // 16):
                tok = idxt_v[s, pl.ds(g * 16, 16)]
                fidx_v[su, pl.ds(g * 16, 16)] = (
                    lax.shift_left(lax.shift_right_logical(tok, 9), 7)
                    + jnp.bitwise_and(tok, jnp.int32(127)))
        gathers = [
            pltpu.async_copy(
                t4_hbm.at[fidx_v.at[su]], grows_v.at[su], gsem[su])
            for su in range(4)
        ]
        writes = []
        for su in range(4):
            s = steps[su]
            f = s // NCH
            c = s % NCH
            gathers[su].wait()
            for g in range(CH // 16):
                tok = idxt_v[s, pl.ds(g * 16, 16)]
                qoff = lax.shift_left(jnp.bitwise_and(
                    lax.shift_right_logical(tok, 7), jnp.int32(3)), 5)
                rowsg = g * 16 + lanes
                for d in range(D):
                    dtile_v[su, pl.ds(d * CH + g * 16, 16)] = \
                        plsc.load_gather(
                            grows_v.at[su], [rowsg, qoff + d])
            for db in range(4):
                writes.append(pltpu.async_copy(
                    dtile_v.at[su, pl.ds(db * 8 * CH, 8 * CH)],
                    out_hbm.at[f, db, blk0 + c],
                    wsem[su]))
        for wcp in writes:
            wcp.wait()
        return 0

    lax.fori_loop(0, NF * NCH // 4, quad, 0)

    def numeric(m, _):
        nwr = []
        for su in range(2):
            u = m * 2 + su
            c = u // NCONT
            n = u % NCONT
            ncol = jnp.full((16,), 1, jnp.int32) * n
            xv = [
                plsc.load_gather(
                    xnum_v, [c * CH + g * 16 + lanes, ncol])
                for g in range(CH // 16)
            ]
            for d in range(D):
                wsp = plsc.load_gather(
                    w_v, [ncol, jnp.full((16,), d, jnp.int32)])
                bsp = plsc.load_gather(
                    b_v, [ncol, jnp.full((16,), d, jnp.int32)])
                for g in range(CH // 16):
                    ntile_v[su, pl.ds(d * CH + g * 16, 16)] = \
                        xv[g] * wsp + bsp
            for db in range(4):
                nwr.append(pltpu.async_copy(
                    ntile_v.at[su, pl.ds(db * 8 * CH, 8 * CH)],
                    out_hbm.at[NF + n, db, blk0 + c],
                    nsem[su]))
        for wcp in nwr:
            wcp.wait()
        return 0

    lax.fori_loop(0, NCH * NCONT // 2, numeric, 0)


_embed = functools.partial(
    pl.kernel,
    out_type=jax.ShapeDtypeStruct((FT, 4, NBLK, 8 * CH), jnp.float32),
    mesh=plsc.VectorSubcoreMesh(core_axis_name="c", subcore_axis_name="s"),
    compiler_params=pltpu.CompilerParams(
        use_tc_tiling_on_sc=False, needs_layout_passes=False
    ),
    scratch_types=[
        pltpu.VMEM((CH, NF), jnp.int32),
        pltpu.VMEM((NF * NCH, CH), jnp.int32),
        pltpu.VMEM((4, CH), jnp.int32),
        pltpu.VMEM((4, CH, 128), jnp.float32),
        pltpu.VMEM((4, D * CH), jnp.float32),
        pltpu.VMEM((RW, NCONT), jnp.float32),
        pltpu.VMEM((NCONT, D), jnp.float32),
        pltpu.VMEM((NCONT, D), jnp.float32),
        pltpu.VMEM((2, D * CH), jnp.float32),
    ] + [pltpu.SemaphoreType.DMA] * 10,
)(_body)


def kernel(x_categ, x_numer, embed_table, num_weights, num_biases):
    t4 = _transpose(embed_table.T)
    out5 = _embed(x_categ.astype(jnp.int32), x_numer, t4,
                  num_weights, num_biases)
    out5 = out5.reshape(FT, 4, NBLK, 8, CH)
    return jnp.transpose(out5, (2, 4, 0, 1, 3)).reshape(B, FT, D)
